# baseline (speedup 1.0000x reference)
import functools

import jax
import jax.numpy as jnp
from jax import lax
from jax.experimental import pallas as pl
from jax.experimental.pallas import tpu as pltpu
from jax.experimental.pallas import tpu_sc as plsc

N = 10000
E = 320000
H = 128

_NC = 2
_NS = 16
_NW = _NC * _NS

_SPLAT_DNUMS = lax.GatherDimensionNumbers(
    offset_dims=(), collapsed_slice_dims=(0,), start_index_map=(0,))


def _splat_lane(vec, l):
    idx = jnp.full((16, 1), l, jnp.int32)
    return lax.gather(vec, idx, _SPLAT_DNUMS, (1,),
                      mode=lax.GatherScatterMode.PROMISE_IN_BOUNDS)


def _mm_kernel(x_ref, w_ref, b_ref, o_ref, *, activation):
    acc = jnp.dot(x_ref[...], w_ref[...], preferred_element_type=jnp.float32)
    acc = acc + b_ref[...]
    if activation == "relu":
        acc = jnp.maximum(acc, 0.0)
    o_ref[...] = acc


def _matmul(x, w_t, b, activation=None, block_m=512):
    m, k = x.shape
    n = w_t.shape[1]
    grid = (pl.cdiv(m, block_m),)
    return pl.pallas_call(
        functools.partial(_mm_kernel, activation=activation),
        grid=grid,
        in_specs=[
            pl.BlockSpec((block_m, k), lambda i: (i, 0)),
            pl.BlockSpec((k, n), lambda i: (0, 0)),
            pl.BlockSpec((1, n), lambda i: (0, 0)),
        ],
        out_specs=pl.BlockSpec((block_m, n), lambda i: (i, 0)),
        out_shape=jax.ShapeDtypeStruct((m, n), jnp.float32),
    )(x, w_t, b.reshape(1, n))


def _mm2_kernel(x_ref, wa_ref, ba_ref, wb_ref, bb_ref, oa_ref, ob_ref):
    x = x_ref[...]
    a = jnp.dot(x, wa_ref[...], preferred_element_type=jnp.float32) + ba_ref[...]
    b = jnp.dot(x, wb_ref[...], preferred_element_type=jnp.float32) + bb_ref[...]
    oa_ref[...] = a
    ob_ref[...] = b


def _matmul2(x, wa_t, ba, wb_t, bb, block_m=512):
    m, k = x.shape
    n = wa_t.shape[1]
    grid = (pl.cdiv(m, block_m),)
    return pl.pallas_call(
        _mm2_kernel,
        grid=grid,
        in_specs=[
            pl.BlockSpec((block_m, k), lambda i: (i, 0)),
            pl.BlockSpec((k, n), lambda i: (0, 0)),
            pl.BlockSpec((1, n), lambda i: (0, 0)),
            pl.BlockSpec((k, n), lambda i: (0, 0)),
            pl.BlockSpec((1, n), lambda i: (0, 0)),
        ],
        out_specs=[
            pl.BlockSpec((block_m, n), lambda i: (i, 0)),
            pl.BlockSpec((block_m, n), lambda i: (i, 0)),
        ],
        out_shape=[
            jax.ShapeDtypeStruct((m, n), jnp.float32),
            jax.ShapeDtypeStruct((m, n), jnp.float32),
        ],
    )(x, wa_t, ba.reshape(1, n), wb_t, bb.reshape(1, n))


def _edge_u(a, b, dst, src):
    per_w = E // _NW
    K = 400
    nchunks = per_w // K
    mesh = plsc.VectorSubcoreMesh(core_axis_name="c", subcore_axis_name="s")

    @functools.partial(
        pl.kernel,
        out_type=jax.ShapeDtypeStruct((E, H), jnp.float32),
        mesh=mesh,
        scratch_types=[
            pltpu.VMEM((K,), jnp.int32),
            pltpu.VMEM((K,), jnp.int32),
            pltpu.VMEM((K, H), jnp.float32),
            pltpu.VMEM((K, H), jnp.float32),
            pltpu.SemaphoreType.DMA,
            pltpu.SemaphoreType.DMA,
        ],
    )
    def k(a_hbm, b_hbm, dst_hbm, src_hbm, u_hbm, didx, sidx, ra, rb, sem_a, sem_b):
        wid = lax.axis_index("s") * _NC + lax.axis_index("c")
        base = wid * per_w

        def chunk(g, carry):
            off = base + g * K
            pltpu.sync_copy(dst_hbm.at[pl.ds(off, K)], didx)
            pltpu.sync_copy(src_hbm.at[pl.ds(off, K)], sidx)
            ca = pltpu.async_copy(a_hbm.at[didx], ra, sem_a)
            cb = pltpu.async_copy(b_hbm.at[sidx], rb, sem_b)
            ca.wait()
            cb.wait()

            def row(i, c2):
                for j in range(8):
                    s = pl.ds(j * 16, 16)
                    ra[i, s] = jnp.maximum(ra[i, s] + rb[i, s], 0.0)
                return c2

            lax.fori_loop(0, K, row, 0)
            pltpu.sync_copy(ra, u_hbm.at[pl.ds(off, K)])
            return carry

        lax.fori_loop(0, nchunks, chunk, 0)

    return k(a, b, dst, src)


def _segment_max(m, dst, out_mode):
    R = 320
    SINK = R
    C = 2000
    G = 128
    NBMAX = (C + G - 1) // G
    nchunks = E // C
    neg_inf = jnp.float32(-jnp.inf)
    mesh = plsc.VectorSubcoreMesh(core_axis_name="c", subcore_axis_name="s")

    @functools.partial(
        pl.kernel,
        out_type=jax.ShapeDtypeStruct((N, H), jnp.float32),
        mesh=mesh,
        compiler_params=pltpu.CompilerParams(needs_layout_passes=False),
        scratch_types=[
            pltpu.VMEM((R + 1, H), jnp.float32),
            pltpu.VMEM((C,), jnp.int32),
            pltpu.VMEM((C + 176, ), jnp.int32),
            pltpu.VMEM((C + 16, ), jnp.int32),
            pltpu.VMEM((2 * G, H), jnp.float32),
            pltpu.SemaphoreType.DMA,
        ],
    )
    def k(m_hbm, dst_hbm, out_hbm, acc, didx, clist, cdst, rows, sem):
        wid = lax.axis_index("s") * _NC + lax.axis_index("c")
        lo = wid * R
        iota = lax.iota(jnp.int32, 16)
        zeros16 = jnp.zeros((16,), jnp.int32)

        def init_row(i, c):
            for j in range(8):
                acc[i, pl.ds(j * 16, 16)] = jnp.full((16,), neg_inf)
            return c

        lax.fori_loop(0, R + 1, init_row, 0)

        def init_clist(i, c):
            clist[pl.ds(i * 16, 16)] = zeros16
            return c

        lax.fori_loop(0, (C + 176) // 16, init_clist, 0)

        def chunk(t, carry):
            chunk_off = t * C
            pltpu.sync_copy(dst_hbm.at[pl.ds(chunk_off, C)], didx)

            def scanv(v, pos):
                local = didx[pl.ds(v * 16, 16)] - lo
                inr = (local >= 0) & (local < R)
                eid = chunk_off + v * 16 + iota
                pfx = jnp.cumsum(inr.astype(jnp.int32))
                ppos = pos + pfx - 1
                plsc.store_scatter(clist, [ppos], eid, mask=inr)
                plsc.store_scatter(cdst, [ppos], local, mask=inr)
                return pos + pfx[15]

            pos = lax.fori_loop(0, C // 16, scanv, 0)

            plsc.store_scatter(cdst, [pos + iota],
                               jnp.full((16,), SINK, jnp.int32))

            ng = (pos + 15) // 16
            nb = (ng + 7) // 8

            def fire(sb):
                boff = (sb % 2) * G
                pltpu.async_copy(
                    m_hbm.at[clist.at[pl.ds(sb * G, G)]],
                    rows.at[pl.ds(boff, G)], sem)

            @pl.when(nb > 0)
            def _():
                fire(0)

            @pl.when(nb > 1)
            def _():
                fire(1)

            def batch(sb, c2):
                pltpu.make_async_copy(
                    m_hbm.at[clist.at[pl.ds(0, G)]],
                    rows.at[pl.ds(0, G)], sem).wait()
                boff = (sb % 2) * G
                g_lo = sb * 8
                g_hi = jnp.minimum(g_lo + 8, ng)

                def group(g, c3):
                    cvec = cdst[pl.ds(g * 16, 16)]
                    rbase = boff + (g - g_lo) * 16
                    for l in range(16):
                        ldv = _splat_lane(cvec, l)
                        for j in range(8):
                            colv = iota + j * 16
                            av = plsc.load_gather(acc, [ldv, colv])
                            mv = rows[rbase + l, pl.ds(j * 16, 16)]
                            plsc.store_scatter(acc, [ldv, colv],
                                               jnp.maximum(av, mv))
                    return c3

                lax.fori_loop(g_lo, g_hi, group, 0)

                @pl.when(sb + 2 < nb)
                def _():
                    fire(sb + 2)

                return c2

            lax.fori_loop(0, nb, batch, 0)
            return carry

        lax.fori_loop(0, nchunks, chunk, 0)

        def fix_row(i, c):
            for j in range(8):
                s = pl.ds(j * 16, 16)
                v = acc[i, s]
                if out_mode == "relu":
                    acc[i, s] = jnp.maximum(v, 0.0)
                else:
                    acc[i, s] = jnp.where(v == neg_inf, 0.0, v)
            return c

        lax.fori_loop(0, R, fix_row, 0)

        last = N - (_NW - 1) * R

        @pl.when(wid < _NW - 1)
        def _():
            pltpu.sync_copy(acc.at[pl.ds(0, R)], out_hbm.at[pl.ds(lo, R)])

        @pl.when(wid == _NW - 1)
        def _():
            pltpu.sync_copy(acc.at[pl.ds(0, last)], out_hbm.at[pl.ds(lo, last)])

    return k(m, dst)


def _edge_conv(z, src, dst, w1, b1, w2, b2, out_mode):
    w1d = w1[:, :H].T
    w1s = w1[:, H:].T
    a_dst, b_src = _matmul2(z, w1d, b1, w1s, jnp.zeros_like(b1))
    u = _edge_u(a_dst, b_src, dst, src)
    m = _matmul(u, w2.T, b2)
    return _segment_max(m, dst, out_mode)


def kernel(x, h, edge_index, enc_w, enc_b, conv0_w1, conv0_b1, conv0_w2, conv0_b2, conv1_w1, conv1_b1, conv1_w2, conv1_b2, dec_w, dec_b, dec_w1, dec_b1, head_w, head_b, term_w, term_b):
    src = edge_index[0]
    dst = edge_index[1]
    z = _matmul(jnp.concatenate([x, h], axis=1), enc_w.T, enc_b, activation="relu")
    hh = _edge_conv(z, src, dst, conv0_w1, conv0_b1, conv0_w2, conv0_b2, "relu")
    hh = _edge_conv(hh, src, dst, conv1_w1, conv1_b1, conv1_w2, conv1_b2, "zero")
    o = _matmul(jnp.concatenate([hh, z], axis=1), dec_w.T, dec_b, activation="relu")
    o = _matmul(o, dec_w1.T, dec_b1, activation="relu")
    y = jax.nn.sigmoid(o @ head_w.T + head_b)
    h_bar = jnp.mean(hh, axis=0)
    t = jax.nn.sigmoid(h_bar @ term_w.T + term_b)
    return (y, t, hh)


# speedup vs baseline: 1.0001x; 1.0001x over previous
import functools

import jax
import jax.numpy as jnp
from jax import lax
from jax.experimental import pallas as pl
from jax.experimental.pallas import tpu as pltpu
from jax.experimental.pallas import tpu_sc as plsc

N = 10000
E = 320000
H = 128

_NC = 2
_NS = 16
_NW = _NC * _NS

_SPLAT_DNUMS = lax.GatherDimensionNumbers(
    offset_dims=(), collapsed_slice_dims=(0,), start_index_map=(0,))


def _splat_lane(vec, l):
    idx = jnp.full((16, 1), l, jnp.int32)
    return lax.gather(vec, idx, _SPLAT_DNUMS, (1,),
                      mode=lax.GatherScatterMode.PROMISE_IN_BOUNDS)


def _mm_kernel(x_ref, w_ref, b_ref, o_ref, *, activation):
    acc = jnp.dot(x_ref[...], w_ref[...], preferred_element_type=jnp.float32)
    acc = acc + b_ref[...]
    if activation == "relu":
        acc = jnp.maximum(acc, 0.0)
    o_ref[...] = acc


def _matmul(x, w_t, b, activation=None, block_m=512):
    m, k = x.shape
    n = w_t.shape[1]
    grid = (pl.cdiv(m, block_m),)
    return pl.pallas_call(
        functools.partial(_mm_kernel, activation=activation),
        grid=grid,
        in_specs=[
            pl.BlockSpec((block_m, k), lambda i: (i, 0)),
            pl.BlockSpec((k, n), lambda i: (0, 0)),
            pl.BlockSpec((1, n), lambda i: (0, 0)),
        ],
        out_specs=pl.BlockSpec((block_m, n), lambda i: (i, 0)),
        out_shape=jax.ShapeDtypeStruct((m, n), jnp.float32),
    )(x, w_t, b.reshape(1, n))


def _mm2_kernel(x_ref, wa_ref, ba_ref, wb_ref, bb_ref, oa_ref, ob_ref):
    x = x_ref[...]
    a = jnp.dot(x, wa_ref[...], preferred_element_type=jnp.float32) + ba_ref[...]
    b = jnp.dot(x, wb_ref[...], preferred_element_type=jnp.float32) + bb_ref[...]
    oa_ref[...] = a
    ob_ref[...] = b


def _matmul2(x, wa_t, ba, wb_t, bb, block_m=512):
    m, k = x.shape
    n = wa_t.shape[1]
    grid = (pl.cdiv(m, block_m),)
    return pl.pallas_call(
        _mm2_kernel,
        grid=grid,
        in_specs=[
            pl.BlockSpec((block_m, k), lambda i: (i, 0)),
            pl.BlockSpec((k, n), lambda i: (0, 0)),
            pl.BlockSpec((1, n), lambda i: (0, 0)),
            pl.BlockSpec((k, n), lambda i: (0, 0)),
            pl.BlockSpec((1, n), lambda i: (0, 0)),
        ],
        out_specs=[
            pl.BlockSpec((block_m, n), lambda i: (i, 0)),
            pl.BlockSpec((block_m, n), lambda i: (i, 0)),
        ],
        out_shape=[
            jax.ShapeDtypeStruct((m, n), jnp.float32),
            jax.ShapeDtypeStruct((m, n), jnp.float32),
        ],
    )(x, wa_t, ba.reshape(1, n), wb_t, bb.reshape(1, n))


def _edge_u(a, b, dst, src):
    per_w = E // _NW
    K = 400
    nchunks = per_w // K
    mesh = plsc.VectorSubcoreMesh(core_axis_name="c", subcore_axis_name="s")

    @functools.partial(
        pl.kernel,
        out_type=jax.ShapeDtypeStruct((E, H), jnp.float32),
        mesh=mesh,
        scratch_types=[
            pltpu.VMEM((K,), jnp.int32),
            pltpu.VMEM((K,), jnp.int32),
            pltpu.VMEM((K, H), jnp.float32),
            pltpu.VMEM((K, H), jnp.float32),
            pltpu.SemaphoreType.DMA,
            pltpu.SemaphoreType.DMA,
        ],
    )
    def k(a_hbm, b_hbm, dst_hbm, src_hbm, u_hbm, didx, sidx, ra, rb, sem_a, sem_b):
        wid = lax.axis_index("s") * _NC + lax.axis_index("c")
        base = wid * per_w

        def chunk(g, carry):
            off = base + g * K
            pltpu.sync_copy(dst_hbm.at[pl.ds(off, K)], didx)
            pltpu.sync_copy(src_hbm.at[pl.ds(off, K)], sidx)
            ca = pltpu.async_copy(a_hbm.at[didx], ra, sem_a)
            cb = pltpu.async_copy(b_hbm.at[sidx], rb, sem_b)
            ca.wait()
            cb.wait()

            def row(i, c2):
                for j in range(8):
                    s = pl.ds(j * 16, 16)
                    ra[i, s] = jnp.maximum(ra[i, s] + rb[i, s], 0.0)
                return c2

            lax.fori_loop(0, K, row, 0)
            pltpu.sync_copy(ra, u_hbm.at[pl.ds(off, K)])
            return carry

        lax.fori_loop(0, nchunks, chunk, 0)

    return k(a, b, dst, src)


def _segment_max(m, dst, out_mode):
    R = 320
    SINK = R
    C = 2000
    G = 128
    NBMAX = (C + G - 1) // G
    nchunks = E // C
    neg_inf = jnp.float32(-jnp.inf)
    mesh = plsc.VectorSubcoreMesh(core_axis_name="c", subcore_axis_name="s")

    @functools.partial(
        pl.kernel,
        out_type=jax.ShapeDtypeStruct((N, H), jnp.float32),
        mesh=mesh,
        compiler_params=pltpu.CompilerParams(needs_layout_passes=False),
        scratch_types=[
            pltpu.VMEM((R + 1, H), jnp.float32),
            pltpu.VMEM((C,), jnp.int32),
            pltpu.VMEM((C + 176, ), jnp.int32),
            pltpu.VMEM((C + 16, ), jnp.int32),
            pltpu.VMEM((2 * G, H), jnp.float32),
            pltpu.SemaphoreType.DMA,
        ],
    )
    def k(m_hbm, dst_hbm, out_hbm, acc, didx, clist, cdst, rows, sem):
        wid = lax.axis_index("s") * _NC + lax.axis_index("c")
        lo = wid * R
        iota = lax.iota(jnp.int32, 16)
        zeros16 = jnp.zeros((16,), jnp.int32)

        def init_row(i, c):
            for j in range(8):
                acc[i, pl.ds(j * 16, 16)] = jnp.full((16,), neg_inf)
            return c

        lax.fori_loop(0, R + 1, init_row, 0)

        def init_clist(i, c):
            clist[pl.ds(i * 16, 16)] = zeros16
            return c

        lax.fori_loop(0, (C + 176) // 16, init_clist, 0)

        def chunk(t, carry):
            chunk_off = t * C
            pltpu.sync_copy(dst_hbm.at[pl.ds(chunk_off, C)], didx)

            def scanv(v, pos):
                local = didx[pl.ds(v * 16, 16)] - lo
                inr = (local >= 0) & (local < R)
                eid = chunk_off + v * 16 + iota
                pfx = jnp.cumsum(inr.astype(jnp.int32))
                ppos = pos + pfx - 1
                plsc.store_scatter(clist, [ppos], eid, mask=inr)
                plsc.store_scatter(cdst, [ppos], local, mask=inr)
                return pos + pfx[15]

            pos = lax.fori_loop(0, C // 16, scanv, 0)

            plsc.store_scatter(cdst, [pos + iota],
                               jnp.full((16,), SINK, jnp.int32))

            ng = (pos + 15) // 16
            nb = (ng + 7) // 8

            def fire(sb):
                boff = (sb % 2) * G
                pltpu.async_copy(
                    m_hbm.at[clist.at[pl.ds(sb * G, G)]],
                    rows.at[pl.ds(boff, G)], sem)

            @pl.when(nb > 0)
            def _():
                fire(0)

            @pl.when(nb > 1)
            def _():
                fire(1)

            def batch(sb, c2):
                pltpu.make_async_copy(
                    m_hbm.at[clist.at[pl.ds(0, G)]],
                    rows.at[pl.ds(0, G)], sem).wait()
                boff = (sb % 2) * G
                g_lo = sb * 8
                g_hi = jnp.minimum(g_lo + 8, ng)

                def group(g, c3):
                    cvec = cdst[pl.ds(g * 16, 16)]
                    rbase = boff + (g - g_lo) * 16
                    for l in range(16):
                        r = cvec[l]
                        for j in range(8):
                            s = pl.ds(j * 16, 16)
                            acc[r, s] = jnp.maximum(acc[r, s],
                                                    rows[rbase + l, s])
                    return c3

                lax.fori_loop(g_lo, g_hi, group, 0)

                @pl.when(sb + 2 < nb)
                def _():
                    fire(sb + 2)

                return c2

            lax.fori_loop(0, nb, batch, 0)
            return carry

        lax.fori_loop(0, nchunks, chunk, 0)

        def fix_row(i, c):
            for j in range(8):
                s = pl.ds(j * 16, 16)
                v = acc[i, s]
                if out_mode == "relu":
                    acc[i, s] = jnp.maximum(v, 0.0)
                else:
                    acc[i, s] = jnp.where(v == neg_inf, 0.0, v)
            return c

        lax.fori_loop(0, R, fix_row, 0)

        last = N - (_NW - 1) * R

        @pl.when(wid < _NW - 1)
        def _():
            pltpu.sync_copy(acc.at[pl.ds(0, R)], out_hbm.at[pl.ds(lo, R)])

        @pl.when(wid == _NW - 1)
        def _():
            pltpu.sync_copy(acc.at[pl.ds(0, last)], out_hbm.at[pl.ds(lo, last)])

    return k(m, dst)


def _edge_conv(z, src, dst, w1, b1, w2, b2, out_mode):
    w1d = w1[:, :H].T
    w1s = w1[:, H:].T
    a_dst, b_src = _matmul2(z, w1d, b1, w1s, jnp.zeros_like(b1))
    u = _edge_u(a_dst, b_src, dst, src)
    m = _matmul(u, w2.T, b2)
    return _segment_max(m, dst, out_mode)


def kernel(x, h, edge_index, enc_w, enc_b, conv0_w1, conv0_b1, conv0_w2, conv0_b2, conv1_w1, conv1_b1, conv1_w2, conv1_b2, dec_w, dec_b, dec_w1, dec_b1, head_w, head_b, term_w, term_b):
    src = edge_index[0]
    dst = edge_index[1]
    z = _matmul(jnp.concatenate([x, h], axis=1), enc_w.T, enc_b, activation="relu")
    hh = _edge_conv(z, src, dst, conv0_w1, conv0_b1, conv0_w2, conv0_b2, "relu")
    hh = _edge_conv(hh, src, dst, conv1_w1, conv1_b1, conv1_w2, conv1_b2, "zero")
    o = _matmul(jnp.concatenate([hh, z], axis=1), dec_w.T, dec_b, activation="relu")
    o = _matmul(o, dec_w1.T, dec_b1, activation="relu")
    y = jax.nn.sigmoid(o @ head_w.T + head_b)
    h_bar = jnp.mean(hh, axis=0)
    t = jax.nn.sigmoid(h_bar @ term_w.T + term_b)
    return (y, t, hh)
